# baseline (device time: 19361 ns/iter reference)
import jax
import jax.numpy as jnp
from jax import lax
from jax.experimental import pallas as pl
from jax.experimental.pallas import tpu as pltpu

N_DEV = 16


def _stage(v, j, k, flip=None):
    n, c = v.shape
    ri = lax.broadcasted_iota(jnp.int32, (n, 1), 0)
    first = (ri & j) == 0
    desc = (ri & k) != 0
    if flip is not None:
        desc = jnp.logical_xor(desc, flip)
    p = jnp.where(first, pltpu.roll(v, n - j, 0), pltpu.roll(v, j, 0))
    lo = jnp.minimum(v, p)
    hi = jnp.maximum(v, p)
    return jnp.where(jnp.logical_xor(first, desc), lo, hi)


def _local_sort(v, flip):
    m, _ = v.shape
    logm = m.bit_length() - 1
    for kk in range(1, logm + 1):
        k = 1 << kk
        j = k >> 1
        while j >= 1:
            v = _stage(v, j, k, flip if k == m else None)
            j >>= 1
    return v


def _merge_stage_packed(w, j, k):
    n, c = w.shape
    ri = lax.broadcasted_iota(jnp.int32, (n, 1), 0)
    li = lax.broadcasted_iota(jnp.int32, (1, c), 1)
    f = ri + jnp.where(li >= 64, 1024, 0)
    first = (f & j) == 0
    desc = (f & k) != 0
    p = jnp.where(first, pltpu.roll(w, n - j, 0), pltpu.roll(w, j, 0))
    lo = jnp.minimum(w, p)
    hi = jnp.maximum(w, p)
    return jnp.where(jnp.logical_xor(first, desc), lo, hi)


def _merge_packed(w):
    for k in (256, 512, 1024):
        j = k >> 1
        while j >= 1:
            w = _merge_stage_packed(w, j, k)
            j >>= 1
    a = w[:, :64]
    b = w[:, 64:]
    w = jnp.concatenate([jnp.minimum(a, b), jnp.maximum(a, b)], axis=1)
    for j in (512, 256, 128):
        w = _merge_stage_packed(w, j, 2048)
    return w


def kernel(x):
    m, n = x.shape

    def body(x_ref, out_ref, gather_ref, w_ref, send_sems, recv_sems,
             ready_sems):
        my = lax.axis_index("i")
        t_my = lax.rem(my, 8)

        barrier_sem = pltpu.get_barrier_semaphore()
        pl.semaphore_signal(barrier_sem, inc=1, device_id=(my,),
                            device_id_type=pl.DeviceIdType.MESH)

        for off in range(1, N_DEV):
            pl.semaphore_signal(
                ready_sems.at[N_DEV - 1 - off], inc=1,
                device_id=(lax.rem(my + off, N_DEV),),
                device_id_type=pl.DeviceIdType.MESH,
            )

        flip = lax.rem(my, 2) != 0
        gather_ref[my] = _local_sort(x_ref[...].astype(jnp.bfloat16), flip)

        pl.semaphore_wait(barrier_sem, 1)

        rdmas = []
        for off in range(1, N_DEV):
            pl.semaphore_wait(ready_sems.at[off - 1], 1)
            rdma = pltpu.make_async_remote_copy(
                src_ref=gather_ref.at[my],
                dst_ref=gather_ref.at[my],
                send_sem=send_sems.at[off - 1],
                recv_sem=recv_sems.at[off - 1],
                device_id=(lax.rem(my + off, N_DEV),),
                device_id_type=pl.DeviceIdType.MESH,
            )
            rdma.start()
            rdmas.append(rdma)
        for rdma in rdmas:
            rdma.wait_send()
        for rdma in rdmas:
            rdma.wait_recv()

        g = gather_ref[...]
        w = jnp.concatenate(
            [g[:8].reshape(8 * m, n), g[8:].reshape(8 * m, n)], axis=1
        )
        w = _merge_packed(w)

        w_ref[...] = w.reshape(8, m, 2 * n)
        wb = w_ref[t_my]
        mine = jnp.where(my < 8, wb[:, :n], wb[:, n:])
        j = 64
        while j >= 1:
            mine = _stage(mine, j, 4096)
            j >>= 1
        out_ref[...] = mine.astype(jnp.float32)

    return pl.pallas_call(
        body,
        out_shape=jax.ShapeDtypeStruct((m, n), jnp.float32),
        in_specs=[pl.BlockSpec(memory_space=pltpu.VMEM)],
        out_specs=pl.BlockSpec(memory_space=pltpu.VMEM),
        scratch_shapes=[
            pltpu.VMEM((N_DEV, m, n), jnp.bfloat16),
            pltpu.VMEM((8, m, 2 * n), jnp.bfloat16),
            pltpu.SemaphoreType.DMA((N_DEV - 1,)),
            pltpu.SemaphoreType.DMA((N_DEV - 1,)),
            pltpu.SemaphoreType.REGULAR((N_DEV - 1,)),
        ],
        compiler_params=pltpu.CompilerParams(collective_id=0),
    )(x)


# device time: 16777 ns/iter; 1.1540x vs baseline; 1.1540x over previous
import jax
import jax.numpy as jnp
from jax import lax
from jax.experimental import pallas as pl
from jax.experimental.pallas import tpu as pltpu

N_DEV = 16


def _stage(v, j, k, flip=None):
    n, c = v.shape
    ri = lax.broadcasted_iota(jnp.int32, (n, 1), 0)
    first = (ri & j) == 0
    desc = (ri & k) != 0
    if flip is not None:
        desc = jnp.logical_xor(desc, flip)
    p = jnp.where(first, pltpu.roll(v, n - j, 0), pltpu.roll(v, j, 0))
    lo = jnp.minimum(v, p)
    hi = jnp.maximum(v, p)
    return jnp.where(jnp.logical_xor(first, desc), lo, hi)


def _local_sort(v, flip):
    m, _ = v.shape
    logm = m.bit_length() - 1
    for kk in range(1, logm + 1):
        k = 1 << kk
        j = k >> 1
        while j >= 1:
            v = _stage(v, j, k, flip if k == m else None)
            j >>= 1
    return v


def _merge_stage_packed(w, j, k):
    n, c = w.shape
    if j >= 16:
        g = n // (2 * j)
        r = w.reshape(g, 2, j, c)
        a = r[:, 0, :, :]
        b = r[:, 1, :, :]
        lo = jnp.minimum(a, b)
        hi = jnp.maximum(a, b)
        if k == 1024:
            li = lax.broadcasted_iota(jnp.int32, (1, 1, c), 2)
            dm = li >= 64
        else:
            gi = lax.broadcasted_iota(jnp.int32, (g, 1, 1), 0)
            dm = ((gi * (2 * j)) & k) != 0
        na = jnp.where(dm, hi, lo)
        nb = jnp.where(dm, lo, hi)
        return jnp.concatenate([na[:, None], nb[:, None]], axis=1).reshape(n, c)
    ri = lax.broadcasted_iota(jnp.int32, (n, 1), 0)
    li = lax.broadcasted_iota(jnp.int32, (1, c), 1)
    f = ri + jnp.where(li >= 64, 1024, 0)
    first = (f & j) == 0
    desc = (f & k) != 0
    p = jnp.where(first, pltpu.roll(w, n - j, 0), pltpu.roll(w, j, 0))
    lo = jnp.minimum(w, p)
    hi = jnp.maximum(w, p)
    return jnp.where(jnp.logical_xor(first, desc), lo, hi)


def _merge_packed(w):
    for k in (256, 512, 1024):
        j = k >> 1
        while j >= 1:
            w = _merge_stage_packed(w, j, k)
            j >>= 1
    a = w[:, :64]
    b = w[:, 64:]
    w = jnp.concatenate([jnp.minimum(a, b), jnp.maximum(a, b)], axis=1)
    for j in (512, 256, 128):
        w = _merge_stage_packed(w, j, 2048)
    return w


def kernel(x):
    m, n = x.shape

    def body(x_ref, out_ref, gather_ref, w_ref, send_sems, recv_sems,
             ready_sems):
        my = lax.axis_index("i")
        t_my = lax.rem(my, 8)

        barrier_sem = pltpu.get_barrier_semaphore()
        pl.semaphore_signal(barrier_sem, inc=1, device_id=(my,),
                            device_id_type=pl.DeviceIdType.MESH)

        for off in range(1, N_DEV):
            pl.semaphore_signal(
                ready_sems.at[N_DEV - 1 - off], inc=1,
                device_id=(lax.rem(my + off, N_DEV),),
                device_id_type=pl.DeviceIdType.MESH,
            )

        flip = lax.rem(my, 2) != 0
        gather_ref[my] = _local_sort(x_ref[...].astype(jnp.bfloat16), flip)

        pl.semaphore_wait(barrier_sem, 1)

        rdmas = []
        for off in range(1, N_DEV):
            pl.semaphore_wait(ready_sems.at[off - 1], 1)
            rdma = pltpu.make_async_remote_copy(
                src_ref=gather_ref.at[my],
                dst_ref=gather_ref.at[my],
                send_sem=send_sems.at[off - 1],
                recv_sem=recv_sems.at[off - 1],
                device_id=(lax.rem(my + off, N_DEV),),
                device_id_type=pl.DeviceIdType.MESH,
            )
            rdma.start()
            rdmas.append(rdma)
        for rdma in rdmas:
            rdma.wait_recv()

        g = gather_ref[...]
        w = jnp.concatenate(
            [g[:8].reshape(8 * m, n), g[8:].reshape(8 * m, n)], axis=1
        )
        w = _merge_packed(w)

        w_ref[...] = w.reshape(8, m, 2 * n)
        for rdma in rdmas:
            rdma.wait_send()
        wb = w_ref[t_my]
        mine = jnp.where(my < 8, wb[:, :n], wb[:, n:])
        j = 64
        while j >= 1:
            mine = _stage(mine, j, 4096)
            j >>= 1
        out_ref[...] = mine.astype(jnp.float32)

    return pl.pallas_call(
        body,
        out_shape=jax.ShapeDtypeStruct((m, n), jnp.float32),
        in_specs=[pl.BlockSpec(memory_space=pltpu.VMEM)],
        out_specs=pl.BlockSpec(memory_space=pltpu.VMEM),
        scratch_shapes=[
            pltpu.VMEM((N_DEV, m, n), jnp.bfloat16),
            pltpu.VMEM((8, m, 2 * n), jnp.bfloat16),
            pltpu.SemaphoreType.DMA((N_DEV - 1,)),
            pltpu.SemaphoreType.DMA((N_DEV - 1,)),
            pltpu.SemaphoreType.REGULAR((N_DEV - 1,)),
        ],
        compiler_params=pltpu.CompilerParams(collective_id=0),
    )(x)


# device time: 16662 ns/iter; 1.1620x vs baseline; 1.0069x over previous
import jax
import jax.numpy as jnp
from jax import lax
from jax.experimental import pallas as pl
from jax.experimental.pallas import tpu as pltpu

N_DEV = 16


def _stage(v, j, k, flip=None):
    n, c = v.shape
    if j >= 16:
        g = n // (2 * j)
        r = v.reshape(g, 2, j, c)
        a = r[:, 0, :, :]
        b = r[:, 1, :, :]
        lo = jnp.minimum(a, b)
        hi = jnp.maximum(a, b)
        gi = lax.broadcasted_iota(jnp.int32, (g, 1, 1), 0)
        dm = ((gi * (2 * j)) & k) != 0
        if flip is not None:
            dm = jnp.logical_xor(dm, flip)
        na = jnp.where(dm, hi, lo)
        nb = jnp.where(dm, lo, hi)
        return jnp.concatenate([na[:, None], nb[:, None]], axis=1).reshape(n, c)
    ri = lax.broadcasted_iota(jnp.int32, (n, 1), 0)
    first = (ri & j) == 0
    desc = (ri & k) != 0
    if flip is not None:
        desc = jnp.logical_xor(desc, flip)
    p = jnp.where(first, pltpu.roll(v, n - j, 0), pltpu.roll(v, j, 0))
    lo = jnp.minimum(v, p)
    hi = jnp.maximum(v, p)
    return jnp.where(jnp.logical_xor(first, desc), lo, hi)


def _local_sort(v, flip):
    m, _ = v.shape
    logm = m.bit_length() - 1
    for kk in range(1, logm + 1):
        k = 1 << kk
        j = k >> 1
        while j >= 1:
            v = _stage(v, j, k, flip if k == m else None)
            j >>= 1
    return v


def _merge_stage_packed(w, j, k):
    n, c = w.shape
    if j >= 16:
        g = n // (2 * j)
        r = w.reshape(g, 2, j, c)
        a = r[:, 0, :, :]
        b = r[:, 1, :, :]
        lo = jnp.minimum(a, b)
        hi = jnp.maximum(a, b)
        if k == 1024:
            li = lax.broadcasted_iota(jnp.int32, (1, 1, c), 2)
            dm = li >= 64
        else:
            gi = lax.broadcasted_iota(jnp.int32, (g, 1, 1), 0)
            dm = ((gi * (2 * j)) & k) != 0
        na = jnp.where(dm, hi, lo)
        nb = jnp.where(dm, lo, hi)
        return jnp.concatenate([na[:, None], nb[:, None]], axis=1).reshape(n, c)
    ri = lax.broadcasted_iota(jnp.int32, (n, 1), 0)
    li = lax.broadcasted_iota(jnp.int32, (1, c), 1)
    f = ri + jnp.where(li >= 64, 1024, 0)
    first = (f & j) == 0
    desc = (f & k) != 0
    p = jnp.where(first, pltpu.roll(w, n - j, 0), pltpu.roll(w, j, 0))
    lo = jnp.minimum(w, p)
    hi = jnp.maximum(w, p)
    return jnp.where(jnp.logical_xor(first, desc), lo, hi)


def _merge_packed(w):
    for k in (256, 512, 1024):
        j = k >> 1
        while j >= 1:
            w = _merge_stage_packed(w, j, k)
            j >>= 1
    p = pltpu.roll(w, 64, 1)
    li = lax.broadcasted_iota(jnp.int32, (1, w.shape[1]), 1)
    w = jnp.where(li < 64, jnp.minimum(w, p), jnp.maximum(w, p))
    for j in (512, 256, 128):
        w = _merge_stage_packed(w, j, 2048)
    return w


def kernel(x):
    m, n = x.shape

    def body(x_ref, out_ref, gather_ref, w_ref, send_sems, recv_sems,
             ready_sems):
        my = lax.axis_index("i")
        t_my = lax.rem(my, 8)

        barrier_sem = pltpu.get_barrier_semaphore()
        pl.semaphore_signal(barrier_sem, inc=1, device_id=(my,),
                            device_id_type=pl.DeviceIdType.MESH)

        for off in range(1, N_DEV):
            pl.semaphore_signal(
                ready_sems.at[N_DEV - 1 - off], inc=1,
                device_id=(lax.rem(my + off, N_DEV),),
                device_id_type=pl.DeviceIdType.MESH,
            )

        flip = lax.rem(my, 2) != 0
        gather_ref[my] = _local_sort(x_ref[...].astype(jnp.bfloat16), flip)

        pl.semaphore_wait(barrier_sem, 1)

        rdmas = []
        for off in range(1, N_DEV):
            pl.semaphore_wait(ready_sems.at[off - 1], 1)
            rdma = pltpu.make_async_remote_copy(
                src_ref=gather_ref.at[my],
                dst_ref=gather_ref.at[my],
                send_sem=send_sems.at[off - 1],
                recv_sem=recv_sems.at[off - 1],
                device_id=(lax.rem(my + off, N_DEV),),
                device_id_type=pl.DeviceIdType.MESH,
            )
            rdma.start()
            rdmas.append(rdma)
        for rdma in rdmas:
            rdma.wait_recv()

        g = gather_ref[...]
        w = jnp.concatenate(
            [g[:8].reshape(8 * m, n), g[8:].reshape(8 * m, n)], axis=1
        )
        w = _merge_packed(w)

        w_ref[...] = w.reshape(8, m, 2 * n)
        for rdma in rdmas:
            rdma.wait_send()
        wb = w_ref[t_my]
        mine = jnp.where(my < 8, wb[:, :n], wb[:, n:])
        j = 64
        while j >= 1:
            mine = _stage(mine, j, 4096)
            j >>= 1
        out_ref[...] = mine.astype(jnp.float32)

    return pl.pallas_call(
        body,
        out_shape=jax.ShapeDtypeStruct((m, n), jnp.float32),
        in_specs=[pl.BlockSpec(memory_space=pltpu.VMEM)],
        out_specs=pl.BlockSpec(memory_space=pltpu.VMEM),
        scratch_shapes=[
            pltpu.VMEM((N_DEV, m, n), jnp.bfloat16),
            pltpu.VMEM((8, m, 2 * n), jnp.bfloat16),
            pltpu.SemaphoreType.DMA((N_DEV - 1,)),
            pltpu.SemaphoreType.DMA((N_DEV - 1,)),
            pltpu.SemaphoreType.REGULAR((N_DEV - 1,)),
        ],
        compiler_params=pltpu.CompilerParams(collective_id=0),
    )(x)


# device time: 16502 ns/iter; 1.1733x vs baseline; 1.0097x over previous
import jax
import jax.numpy as jnp
from jax import lax
from jax.experimental import pallas as pl
from jax.experimental.pallas import tpu as pltpu

N_DEV = 16


def _stage(v, j, k, flip=None):
    n, c = v.shape
    if j >= 16:
        g = n // (2 * j)
        r = v.reshape(g, 2, j, c)
        a = r[:, 0, :, :]
        b = r[:, 1, :, :]
        lo = jnp.minimum(a, b)
        hi = jnp.maximum(a, b)
        gi = lax.broadcasted_iota(jnp.int32, (g, 1, 1), 0)
        dm = ((gi * (2 * j)) & k) != 0
        if flip is not None:
            dm = jnp.logical_xor(dm, flip)
        na = jnp.where(dm, hi, lo)
        nb = jnp.where(dm, lo, hi)
        return jnp.concatenate([na[:, None], nb[:, None]], axis=1).reshape(n, c)
    ri = lax.broadcasted_iota(jnp.int32, (n, 1), 0)
    first = (ri & j) == 0
    desc = (ri & k) != 0
    if flip is not None:
        desc = jnp.logical_xor(desc, flip)
    p = jnp.where(first, pltpu.roll(v, n - j, 0), pltpu.roll(v, j, 0))
    lo = jnp.minimum(v, p)
    hi = jnp.maximum(v, p)
    return jnp.where(jnp.logical_xor(first, desc), lo, hi)


def _local_sort(v, flip):
    m, _ = v.shape
    logm = m.bit_length() - 1
    for kk in range(1, logm + 1):
        k = 1 << kk
        j = k >> 1
        while j >= 1:
            v = _stage(v, j, k, flip if k == m else None)
            j >>= 1
    return v


def _merge_stage_packed(w, j, k):
    n, c = w.shape
    if j >= 16:
        g = n // (2 * j)
        r = w.reshape(g, 2, j, c)
        a = r[:, 0, :, :]
        b = r[:, 1, :, :]
        lo = jnp.minimum(a, b)
        hi = jnp.maximum(a, b)
        if k == 1024:
            li = lax.broadcasted_iota(jnp.int32, (1, 1, c), 2)
            dm = li >= 64
        else:
            gi = lax.broadcasted_iota(jnp.int32, (g, 1, 1), 0)
            dm = ((gi * (2 * j)) & k) != 0
        na = jnp.where(dm, hi, lo)
        nb = jnp.where(dm, lo, hi)
        return jnp.concatenate([na[:, None], nb[:, None]], axis=1).reshape(n, c)
    ri = lax.broadcasted_iota(jnp.int32, (n, 1), 0)
    li = lax.broadcasted_iota(jnp.int32, (1, c), 1)
    f = ri + jnp.where(li >= 64, 1024, 0)
    first = (f & j) == 0
    desc = (f & k) != 0
    p = jnp.where(first, pltpu.roll(w, n - j, 0), pltpu.roll(w, j, 0))
    lo = jnp.minimum(w, p)
    hi = jnp.maximum(w, p)
    return jnp.where(jnp.logical_xor(first, desc), lo, hi)


def _merge_packed(w):
    for k in (256, 512, 1024):
        j = k >> 1
        while j >= 1:
            w = _merge_stage_packed(w, j, k)
            j >>= 1
    p = pltpu.roll(w, 64, 1)
    li = lax.broadcasted_iota(jnp.int32, (1, w.shape[1]), 1)
    w = jnp.where(li < 64, jnp.minimum(w, p), jnp.maximum(w, p))
    for j in (512, 256, 128):
        w = _merge_stage_packed(w, j, 2048)
    return w


def kernel(x):
    m, n = x.shape

    def body(x_ref, out_ref, gather_ref, w_ref, send_sems, recv_sems):
        my = lax.axis_index("i")
        t_my = lax.rem(my, 8)

        barrier_sem = pltpu.get_barrier_semaphore()
        for off in range(1, N_DEV):
            pl.semaphore_signal(
                barrier_sem, inc=1,
                device_id=(lax.rem(my + off, N_DEV),),
                device_id_type=pl.DeviceIdType.MESH,
            )

        flip = lax.rem(my, 2) != 0
        gather_ref[my] = _local_sort(x_ref[...].astype(jnp.bfloat16), flip)

        pl.semaphore_wait(barrier_sem, N_DEV - 1)

        rdmas = []
        for off in range(1, N_DEV):
            rdma = pltpu.make_async_remote_copy(
                src_ref=gather_ref.at[my],
                dst_ref=gather_ref.at[my],
                send_sem=send_sems.at[off - 1],
                recv_sem=recv_sems.at[off - 1],
                device_id=(lax.rem(my + off, N_DEV),),
                device_id_type=pl.DeviceIdType.MESH,
            )
            rdma.start()
            rdmas.append(rdma)
        for rdma in rdmas:
            rdma.wait_recv()

        g = gather_ref[...]
        w = jnp.concatenate(
            [g[:8].reshape(8 * m, n), g[8:].reshape(8 * m, n)], axis=1
        )
        w = _merge_packed(w)

        w_ref[...] = w.reshape(8, m, 2 * n)
        for rdma in rdmas:
            rdma.wait_send()
        wb = w_ref[t_my]
        mine = jnp.where(my < 8, wb[:, :n], wb[:, n:])
        j = 64
        while j >= 1:
            mine = _stage(mine, j, 4096)
            j >>= 1
        out_ref[...] = mine.astype(jnp.float32)

    return pl.pallas_call(
        body,
        out_shape=jax.ShapeDtypeStruct((m, n), jnp.float32),
        in_specs=[pl.BlockSpec(memory_space=pltpu.VMEM)],
        out_specs=pl.BlockSpec(memory_space=pltpu.VMEM),
        scratch_shapes=[
            pltpu.VMEM((N_DEV, m, n), jnp.bfloat16),
            pltpu.VMEM((8, m, 2 * n), jnp.bfloat16),
            pltpu.SemaphoreType.DMA((N_DEV - 1,)),
            pltpu.SemaphoreType.DMA((N_DEV - 1,)),
        ],
        compiler_params=pltpu.CompilerParams(collective_id=0),
    )(x)
